# Initial kernel scaffold; baseline (speedup 1.0000x reference)
import jax
import jax.numpy as jnp
from jax.experimental import pallas as pl


def kernel(x, edge_index, gamma, beta, W):
    raise NotImplementedError("write your pallas kernel here")


# speedup vs baseline: 4.6011x; 4.6011x over previous
import functools

import jax
import jax.numpy as jnp
from jax import lax
from jax.experimental import pallas as pl
from jax.experimental.pallas import tpu as pltpu
from jax.experimental.pallas import tpu_sc as plsc

N_NODES = 10000
N_EDGES = 160000
D_IN = 256
D_OUT = 256
DH = 128

NC = 2
NS = 16
EPT = N_EDGES // NS
KB = 128
NFULL = EPT // KB
TAIL = EPT - NFULL * KB
ROWS_PT = 624
ROWS_TAIL = N_NODES - NS * ROWS_PT


def _bn_mm_body(x_ref, g_ref, b_ref, w_ref, hl_ref, hr_ref):
    x = x_ref[...]
    mean = jnp.mean(x, axis=0, keepdims=True)
    var = jnp.mean((x - mean) * (x - mean), axis=0, keepdims=True)
    scale = g_ref[...] * lax.rsqrt(var + 1e-5)
    xn = (x - mean) * scale + b_ref[...]
    h = lax.dot_general(xn, w_ref[...], (((1,), (1,)), ((), ())),
                        preferred_element_type=jnp.float32)
    hl_ref[...] = h[:, :DH]
    hr_ref[...] = h[:, DH:]


def _relu_body(l_ref, r_ref, o_ref):
    o_ref[:, :DH] = jnp.maximum(l_ref[...], 0.0)
    o_ref[:, DH:] = jnp.maximum(r_ref[...], 0.0)


def _sc_body(hl, hr, src_hbm, dst_hbm, zrows, outl, outr,
             acc, idx_s, idx_d, rows, idx_st, idx_dt, rows_t, sem):
    cid = lax.axis_index("c")
    sid = lax.axis_index("s")

    pltpu.sync_copy(zrows, acc.at[pl.ds(sid * ROWS_PT, ROWS_PT)])

    @pl.when(sid == 0)
    def _():
        pltpu.sync_copy(zrows.at[pl.ds(0, ROWS_TAIL)],
                        acc.at[pl.ds(NS * ROWS_PT, ROWS_TAIL)])

    plsc.subcore_barrier()

    base0 = sid * EPT

    def edge_pass(h_hbm):
        def block(j, _):
            b = base0 + j * KB
            pltpu.sync_copy(src_hbm.at[pl.ds(b, KB)], idx_s)
            pltpu.sync_copy(dst_hbm.at[pl.ds(b, KB)], idx_d)
            pltpu.async_copy(h_hbm.at[idx_s], rows, sem).wait()
            pltpu.sync_copy(rows, acc.at[idx_d], add=True)
            return _
        lax.fori_loop(0, NFULL, block, None)
        bt = base0 + NFULL * KB
        pltpu.sync_copy(src_hbm.at[pl.ds(bt, TAIL)], idx_st)
        pltpu.sync_copy(dst_hbm.at[pl.ds(bt, TAIL)], idx_dt)
        pltpu.async_copy(h_hbm.at[idx_st], rows_t, sem).wait()
        pltpu.sync_copy(rows_t, acc.at[idx_dt], add=True)

    @pl.when(cid == 0)
    def _():
        edge_pass(hl)

    @pl.when(cid == 1)
    def _():
        edge_pass(hr)

    plsc.subcore_barrier()

    r0 = sid * ROWS_PT

    @pl.when(cid == 0)
    def _():
        pltpu.sync_copy(acc.at[pl.ds(r0, ROWS_PT)], outl.at[pl.ds(r0, ROWS_PT)])

        @pl.when(sid == 0)
        def _():
            pltpu.sync_copy(acc.at[pl.ds(NS * ROWS_PT, ROWS_TAIL)],
                            outl.at[pl.ds(NS * ROWS_PT, ROWS_TAIL)])

    @pl.when(cid == 1)
    def _():
        pltpu.sync_copy(acc.at[pl.ds(r0, ROWS_PT)], outr.at[pl.ds(r0, ROWS_PT)])

        @pl.when(sid == 0)
        def _():
            pltpu.sync_copy(acc.at[pl.ds(NS * ROWS_PT, ROWS_TAIL)],
                            outr.at[pl.ds(NS * ROWS_PT, ROWS_TAIL)])


_sc_scatter = functools.partial(
    pl.kernel,
    out_type=(
        jax.ShapeDtypeStruct((N_NODES, DH), jnp.float32),
        jax.ShapeDtypeStruct((N_NODES, DH), jnp.float32),
    ),
    mesh=plsc.VectorSubcoreMesh(core_axis_name="c", subcore_axis_name="s",
                                num_cores=NC, num_subcores=NS),
    scratch_types=[
        pltpu.VMEM_SHARED((N_NODES, DH), jnp.float32),
        pltpu.VMEM((KB,), jnp.int32),
        pltpu.VMEM((KB,), jnp.int32),
        pltpu.VMEM((KB, DH), jnp.float32),
        pltpu.VMEM((TAIL,), jnp.int32),
        pltpu.VMEM((TAIL,), jnp.int32),
        pltpu.VMEM((TAIL, DH), jnp.float32),
        pltpu.SemaphoreType.DMA,
    ],
)(_sc_body)


@jax.jit
def kernel(x, edge_index, gamma, beta, W):
    hl, hr = pl.pallas_call(
        _bn_mm_body,
        out_shape=(
            jax.ShapeDtypeStruct((N_NODES, DH), jnp.float32),
            jax.ShapeDtypeStruct((N_NODES, DH), jnp.float32),
        ),
    )(x, gamma.reshape(1, D_IN), beta.reshape(1, D_IN), W)

    src = edge_index[0]
    dst = edge_index[1]
    zrows = jnp.zeros((ROWS_PT, DH), jnp.float32)
    outl, outr = _sc_scatter(hl, hr, src, dst, zrows)

    out = pl.pallas_call(
        _relu_body,
        out_shape=jax.ShapeDtypeStruct((N_NODES, D_OUT), jnp.float32),
    )(outl, outr)
    return out
